# Initial kernel scaffold; baseline (speedup 1.0000x reference)
import jax
import jax.numpy as jnp
from jax.experimental import pallas as pl


def kernel(x, edge_index, W, b):
    raise NotImplementedError("write your pallas kernel here")


# speedup vs baseline: 17.4312x; 17.4312x over previous
import functools

import jax
import jax.numpy as jnp
from jax import lax
from jax.experimental import pallas as pl
from jax.experimental.pallas import tpu as pltpu
from jax.experimental.pallas import tpu_sc as plsc

N = 10000
NP = 10240
D = 128
DH = 64
E = 320000
CH = 128
GE = 16
NG = 10
NCH = GE * NG
EP = 16 * NCH * CH
RPT = NP // 16
RB = 128
NB = RPT // RB


def _sc_propagate(x2, srcb, dstb):
    mesh = plsc.VectorSubcoreMesh(core_axis_name="c", subcore_axis_name="s")

    @functools.partial(
        pl.kernel,
        out_type=jax.ShapeDtypeStruct((2, NP, DH), jnp.float32),
        mesh=mesh,
        compiler_params=pltpu.CompilerParams(use_tc_tiling_on_sc=False),
        scratch_types=[
            pltpu.VMEM((GE, CH), jnp.int32),
            pltpu.VMEM((GE, CH), jnp.int32),
            pltpu.VMEM((CH, DH), jnp.float32),
            pltpu.VMEM((RB, DH), jnp.float32),
            pltpu.VMEM((RPT, 16), jnp.float32),
            pltpu.VMEM((CH, 16), jnp.float32),
            pltpu.VMEM_SHARED((NP, DH), jnp.float32),
            pltpu.VMEM_SHARED((NP, DH), jnp.float32),
            pltpu.VMEM_SHARED((NP, 16), jnp.float32),
        ],
    )
    def k(x2_h, src_h, dst_h, out_h, src_v, dst_v, rows_v, buf,
          dinv_l, ones_v, y_sh, acc_sh, deg_sh):
        c = lax.axis_index("c")
        s = lax.axis_index("s")
        row0 = s * RPT
        own = pl.ds(row0, RPT)

        z16 = jnp.zeros((16,), jnp.float32)
        o16 = jnp.ones((16,), jnp.float32)

        def z_body(r, carry):
            dinv_l[r] = z16
            return carry

        lax.fori_loop(0, RPT, z_body, 0)

        def o_body(r, carry):
            ones_v[r] = o16
            return carry

        lax.fori_loop(0, CH, o_body, 0)
        pltpu.sync_copy(dinv_l, deg_sh.at[own])
        pltpu.sync_copy(x2_h.at[c, own], y_sh.at[own])
        plsc.subcore_barrier()

        def deg_group(g, carry):
            pltpu.sync_copy(dst_h.at[s, pl.ds(g * GE, GE)], dst_v)

            def deg_body(j, carry2):
                pltpu.sync_copy(ones_v, deg_sh.at[dst_v.at[j]], add=True)
                return carry2

            lax.fori_loop(0, GE, deg_body, 0)
            return carry

        lax.fori_loop(0, NG, deg_group, 0)
        plsc.subcore_barrier()

        pltpu.sync_copy(deg_sh.at[own], dinv_l)

        def dinv_body(r, carry):
            d = dinv_l[r] + 1.0
            i = lax.bitcast_convert_type(d, jnp.int32)
            i = jnp.int32(0x5F3759DF) - (i >> 1)
            q = lax.bitcast_convert_type(i, jnp.float32)
            q = q * (1.5 - 0.5 * d * q * q)
            q = q * (1.5 - 0.5 * d * q * q)
            q = q * (1.5 - 0.5 * d * q * q)
            dinv_l[r] = q
            return carry

        lax.fori_loop(0, RPT, dinv_body, 0)

        def scale_rows(power, to_out):
            def chunk_body(cb, carry):
                rbase = row0 + cb * RB
                sl = pl.ds(rbase, RB)
                if to_out:
                    src_ref = acc_sh.at[sl]
                else:
                    src_ref = y_sh.at[sl] if power == 1 else acc_sh.at[sl]
                pltpu.sync_copy(src_ref, buf)

                def row_body(r, carry2):
                    dv = dinv_l[cb * RB + r]
                    if power == 2:
                        dv = dv * dv
                    for jj in range(DH // 16):
                        csl = pl.ds(jj * 16, 16)
                        buf[r, csl] = buf[r, csl] * dv
                    return carry2

                lax.fori_loop(0, RB, row_body, 0)
                if to_out:
                    pltpu.sync_copy(buf, y_sh.at[sl])
                    pltpu.sync_copy(y_sh.at[sl], out_h.at[c, sl])
                else:
                    pltpu.sync_copy(buf, y_sh.at[sl])
                    pltpu.sync_copy(buf, acc_sh.at[sl])
                return carry

            lax.fori_loop(0, NB, chunk_body, 0)

        scale_rows(1, False)
        plsc.subcore_barrier()

        def hop_group(g, carry):
            pltpu.sync_copy(src_h.at[s, pl.ds(g * GE, GE)], src_v)
            pltpu.sync_copy(dst_h.at[s, pl.ds(g * GE, GE)], dst_v)

            def hop_body(j, carry2):
                pltpu.sync_copy(y_sh.at[src_v.at[j]], rows_v)
                pltpu.sync_copy(rows_v, acc_sh.at[dst_v.at[j]], add=True)
                return carry2

            lax.fori_loop(0, GE, hop_body, 0)
            return carry

        lax.fori_loop(0, NG, hop_group, 0)
        plsc.subcore_barrier()

        scale_rows(2, False)
        plsc.subcore_barrier()

        lax.fori_loop(0, NG, hop_group, 0)
        plsc.subcore_barrier()

        scale_rows(1, True)
    return k(x2, srcb, dstb)


def _tc_linear(h2, W, b):
    def body(h2_ref, w_ref, b_ref, o_ref):
        h = jnp.concatenate([h2_ref[0], h2_ref[1]], axis=1)[:N]
        o_ref[...] = lax.dot_general(
            h, w_ref[...], (((1,), (1,)), ((), ())),
            preferred_element_type=jnp.float32) + b_ref[...]

    return pl.pallas_call(
        body,
        out_shape=jax.ShapeDtypeStruct((N, D), jnp.float32),
    )(h2, W, b)


@jax.jit
def kernel(x, edge_index, W, b):
    src = edge_index[0].astype(jnp.int32)
    dst = edge_index[1].astype(jnp.int32)
    pad = jnp.full((EP - E,), N, jnp.int32)
    srcb = jnp.concatenate([src, pad]).reshape(16, NCH, CH)
    dstb = jnp.concatenate([dst, pad]).reshape(16, NCH, CH)
    xp = jnp.concatenate([x, jnp.zeros((NP - N, D), x.dtype)])
    x2 = xp.reshape(NP, 2, DH).transpose(1, 0, 2)
    h2 = _sc_propagate(x2, srcb, dstb)
    return _tc_linear(h2, W, b)


# speedup vs baseline: 21.3186x; 1.2230x over previous
import functools

import jax
import jax.numpy as jnp
from jax import lax
from jax.experimental import pallas as pl
from jax.experimental.pallas import tpu as pltpu
from jax.experimental.pallas import tpu_sc as plsc

N = 10000
NP = 10240
D = 128
DH = 64
E = 320000
CH = 128
GE = 16
NG = 10
NCH = GE * NG
EP = 16 * NCH * CH
RPT = NP // 16
RB = 64
NB = RPT // RB


def _sc_propagate(x2, srcb, dstb):
    mesh = plsc.VectorSubcoreMesh(core_axis_name="c", subcore_axis_name="s")

    @functools.partial(
        pl.kernel,
        out_type=jax.ShapeDtypeStruct((2, NP, DH), jnp.float32),
        mesh=mesh,
        compiler_params=pltpu.CompilerParams(use_tc_tiling_on_sc=False),
        scratch_types=[
            pltpu.VMEM((GE, CH), jnp.int32),
            pltpu.VMEM((GE, CH), jnp.int32),
            pltpu.VMEM((CH, DH), jnp.float32),
            pltpu.VMEM((CH, DH), jnp.float32),
            pltpu.VMEM((RB, DH), jnp.float32),
            pltpu.VMEM((RPT, 16), jnp.float32),
            pltpu.VMEM((CH, 16), jnp.float32),
            pltpu.VMEM_SHARED((NP, DH), jnp.float32),
            pltpu.VMEM_SHARED((NP, DH), jnp.float32),
            pltpu.VMEM_SHARED((NP, 16), jnp.float32),
            pltpu.SemaphoreType.DMA,
            pltpu.SemaphoreType.DMA,
        ],
    )
    def k(x2_h, src_h, dst_h, out_h, src_v, dst_v, rows_a, rows_b, buf,
          dinv_l, ones_v, y_sh, acc_sh, deg_sh, sem_g, sem_s):
        c = lax.axis_index("c")
        s = lax.axis_index("s")
        row0 = s * RPT
        own = pl.ds(row0, RPT)

        z16 = jnp.zeros((16,), jnp.float32)
        o16 = jnp.ones((16,), jnp.float32)

        def z_body(r, carry):
            dinv_l[r] = z16
            return carry

        lax.fori_loop(0, RPT, z_body, 0)

        def o_body(r, carry):
            ones_v[r] = o16
            return carry

        lax.fori_loop(0, CH, o_body, 0)
        pltpu.sync_copy(dinv_l, deg_sh.at[own])
        pltpu.sync_copy(x2_h.at[c, own], y_sh.at[own])
        plsc.subcore_barrier()

        def deg_group(g, carry):
            pltpu.sync_copy(dst_h.at[s, pl.ds(g * GE, GE)], dst_v)

            def deg_fire(j, carry2):
                pltpu.async_copy(ones_v, deg_sh.at[dst_v.at[j]], sem_s,
                                 add=True)
                return carry2

            lax.fori_loop(0, GE, deg_fire, 0)

            def deg_drain(j, carry2):
                pltpu.make_async_copy(
                    ones_v, deg_sh.at[dst_v.at[0]], sem_s).wait()
                return carry2

            lax.fori_loop(0, GE, deg_drain, 0)
            return carry

        lax.fori_loop(0, NG, deg_group, 0)
        plsc.subcore_barrier()

        pltpu.sync_copy(deg_sh.at[own], dinv_l)

        def dinv_body(r, carry):
            d = dinv_l[r] + 1.0
            i = lax.bitcast_convert_type(d, jnp.int32)
            i = jnp.int32(0x5F3759DF) - (i >> 1)
            q = lax.bitcast_convert_type(i, jnp.float32)
            q = q * (1.5 - 0.5 * d * q * q)
            q = q * (1.5 - 0.5 * d * q * q)
            q = q * (1.5 - 0.5 * d * q * q)
            dinv_l[r] = q
            return carry

        lax.fori_loop(0, RPT, dinv_body, 0)

        def scale_rows(power, to_out):
            def chunk_body(cb, carry):
                rbase = row0 + cb * RB
                sl = pl.ds(rbase, RB)
                if to_out:
                    src_ref = acc_sh.at[sl]
                else:
                    src_ref = y_sh.at[sl] if power == 1 else acc_sh.at[sl]
                pltpu.sync_copy(src_ref, buf)

                def row_body(r, carry2):
                    dv = dinv_l[cb * RB + r]
                    if power == 2:
                        dv = dv * dv
                    for jj in range(DH // 16):
                        csl = pl.ds(jj * 16, 16)
                        buf[r, csl] = buf[r, csl] * dv
                    return carry2

                lax.fori_loop(0, RB, row_body, 0)
                if to_out:
                    pltpu.sync_copy(buf, y_sh.at[sl])
                    pltpu.sync_copy(y_sh.at[sl], out_h.at[c, sl])
                else:
                    pltpu.sync_copy(buf, y_sh.at[sl])
                    pltpu.sync_copy(buf, acc_sh.at[sl])
                return carry

            lax.fori_loop(0, NB, chunk_body, 0)

        scale_rows(1, False)
        plsc.subcore_barrier()

        def g_start(j, dst_buf):
            pltpu.async_copy(y_sh.at[src_v.at[j]], dst_buf, sem_g)

        def g_wait(dst_buf):
            pltpu.make_async_copy(y_sh.at[src_v.at[0]], dst_buf, sem_g).wait()

        def s_start(j, src_buf):
            pltpu.async_copy(src_buf, acc_sh.at[dst_v.at[j]], sem_s, add=True)

        def s_wait(src_buf):
            pltpu.make_async_copy(
                src_buf, acc_sh.at[dst_v.at[0]], sem_s).wait()

        def hop_group(g, carry):
            pltpu.sync_copy(src_h.at[s, pl.ds(g * GE, GE)], src_v)
            pltpu.sync_copy(dst_h.at[s, pl.ds(g * GE, GE)], dst_v)
            g_start(0, rows_a)

            def pair(m, carry2):
                j0 = 2 * m
                g_wait(rows_a)
                g_start(j0 + 1, rows_b)
                s_start(j0, rows_a)
                g_wait(rows_b)
                s_wait(rows_a)

                @pl.when(m < GE // 2 - 1)
                def _():
                    g_start(j0 + 2, rows_a)

                s_start(j0 + 1, rows_b)
                s_wait(rows_b)
                return carry2

            lax.fori_loop(0, GE // 2, pair, 0)
            return carry

        lax.fori_loop(0, NG, hop_group, 0)
        plsc.subcore_barrier()

        scale_rows(2, False)
        plsc.subcore_barrier()

        lax.fori_loop(0, NG, hop_group, 0)
        plsc.subcore_barrier()

        scale_rows(1, True)
    return k(x2, srcb, dstb)


def _tc_linear(h2, W, b):
    def body(h2_ref, w_ref, b_ref, o_ref):
        h = jnp.concatenate([h2_ref[0], h2_ref[1]], axis=1)[:N]
        o_ref[...] = lax.dot_general(
            h, w_ref[...], (((1,), (1,)), ((), ())),
            preferred_element_type=jnp.float32) + b_ref[...]

    return pl.pallas_call(
        body,
        out_shape=jax.ShapeDtypeStruct((N, D), jnp.float32),
    )(h2, W, b)


@jax.jit
def kernel(x, edge_index, W, b):
    src = edge_index[0].astype(jnp.int32)
    dst = edge_index[1].astype(jnp.int32)
    pad = jnp.full((EP - E,), N, jnp.int32)
    srcb = jnp.concatenate([src, pad]).reshape(16, NCH, CH)
    dstb = jnp.concatenate([dst, pad]).reshape(16, NCH, CH)
    xp = jnp.concatenate([x, jnp.zeros((NP - N, D), x.dtype)])
    x2 = xp.reshape(NP, 2, DH).transpose(1, 0, 2)
    h2 = _sc_propagate(x2, srcb, dstb)
    return _tc_linear(h2, W, b)


# speedup vs baseline: 23.5117x; 1.1029x over previous
import functools

import jax
import jax.numpy as jnp
from jax import lax
from jax.experimental import pallas as pl
from jax.experimental.pallas import tpu as pltpu
from jax.experimental.pallas import tpu_sc as plsc

N = 10000
NP = 10240
D = 128
DH = 64
E = 320000
CH = 128
GE = 18
NG = 9
NCH = GE * NG
EP = 16 * NCH * CH
RPT = NP // 16
RB = 64
NB = RPT // RB
NT = GE // 3


def _sc_propagate(x2, srcb, dstb):
    mesh = plsc.VectorSubcoreMesh(core_axis_name="c", subcore_axis_name="s")

    @functools.partial(
        pl.kernel,
        out_type=jax.ShapeDtypeStruct((2, NP, DH), jnp.float32),
        mesh=mesh,
        compiler_params=pltpu.CompilerParams(use_tc_tiling_on_sc=False),
        scratch_types=[
            pltpu.VMEM((GE, CH), jnp.int32),
            pltpu.VMEM((GE, CH), jnp.int32),
            pltpu.VMEM((CH, DH), jnp.float32),
            pltpu.VMEM((CH, DH), jnp.float32),
            pltpu.VMEM((CH, DH), jnp.float32),
            pltpu.VMEM((RB, DH), jnp.float32),
            pltpu.VMEM((RB, 16), jnp.float32),
            pltpu.VMEM((CH, 16), jnp.float32),
            pltpu.VMEM_SHARED((NP, DH), jnp.float32),
            pltpu.VMEM_SHARED((NP, DH), jnp.float32),
            pltpu.VMEM_SHARED((NP, 16), jnp.float32),
            pltpu.SemaphoreType.DMA,
            pltpu.SemaphoreType.DMA,
        ],
    )
    def k(x2_h, src_h, dst_h, out_h, src_v, dst_v, rows_a, rows_b, rows_c,
          buf, dloc, ones_v, y_sh, acc_sh, deg_sh, sem_g, sem_s):
        c = lax.axis_index("c")
        s = lax.axis_index("s")
        row0 = s * RPT
        own = pl.ds(row0, RPT)
        rings = (rows_a, rows_b, rows_c)

        z16 = jnp.zeros((16,), jnp.float32)
        o16 = jnp.ones((16,), jnp.float32)

        def z_body(r, carry):
            dloc[r] = z16
            return carry

        lax.fori_loop(0, RB, z_body, 0)

        def o_body(r, carry):
            ones_v[r] = o16
            return carry

        lax.fori_loop(0, CH, o_body, 0)

        def zc_body(cb, carry):
            pltpu.sync_copy(dloc, deg_sh.at[pl.ds(row0 + cb * RB, RB)])
            return carry

        lax.fori_loop(0, NB, zc_body, 0)
        pltpu.sync_copy(x2_h.at[c, own], y_sh.at[own])
        plsc.subcore_barrier()

        def deg_group(g, carry):
            pltpu.sync_copy(dst_h.at[s, pl.ds(g * GE, GE)], dst_v)

            def deg_fire(j, carry2):
                pltpu.async_copy(ones_v, deg_sh.at[dst_v.at[j]], sem_s,
                                 add=True)
                return carry2

            lax.fori_loop(0, GE, deg_fire, 0)

            def deg_drain(j, carry2):
                pltpu.make_async_copy(
                    ones_v, deg_sh.at[dst_v.at[0]], sem_s).wait()
                return carry2

            lax.fori_loop(0, GE, deg_drain, 0)
            return carry

        lax.fori_loop(0, NG, deg_group, 0)
        plsc.subcore_barrier()

        def dinv_chunk(cb, carry):
            sl = pl.ds(row0 + cb * RB, RB)
            pltpu.sync_copy(deg_sh.at[sl], dloc)

            def dinv_body(r, carry2):
                d = dloc[r] + 1.0
                i = lax.bitcast_convert_type(d, jnp.int32)
                i = jnp.int32(0x5F3759DF) - (i >> 1)
                q = lax.bitcast_convert_type(i, jnp.float32)
                q = q * (1.5 - 0.5 * d * q * q)
                q = q * (1.5 - 0.5 * d * q * q)
                q = q * (1.5 - 0.5 * d * q * q)
                dloc[r] = q
                return carry2

            lax.fori_loop(0, RB, dinv_body, 0)
            pltpu.sync_copy(dloc, deg_sh.at[sl])
            return carry

        lax.fori_loop(0, NB, dinv_chunk, 0)

        def scale_rows(power, to_out):
            def chunk_body(cb, carry):
                sl = pl.ds(row0 + cb * RB, RB)
                if to_out:
                    src_ref = acc_sh.at[sl]
                else:
                    src_ref = y_sh.at[sl] if power == 1 else acc_sh.at[sl]
                pltpu.sync_copy(src_ref, buf)
                pltpu.sync_copy(deg_sh.at[sl], dloc)

                def row_body(r, carry2):
                    dv = dloc[r]
                    if power == 2:
                        dv = dv * dv
                    for jj in range(DH // 16):
                        csl = pl.ds(jj * 16, 16)
                        buf[r, csl] = buf[r, csl] * dv
                    return carry2

                lax.fori_loop(0, RB, row_body, 0)
                if to_out:
                    pltpu.sync_copy(buf, y_sh.at[sl])
                    pltpu.sync_copy(y_sh.at[sl], out_h.at[c, sl])
                else:
                    pltpu.sync_copy(buf, y_sh.at[sl])
                    pltpu.sync_copy(buf, acc_sh.at[sl])
                return carry

            lax.fori_loop(0, NB, chunk_body, 0)

        scale_rows(1, False)
        plsc.subcore_barrier()

        def g_start(j, dst_buf):
            pltpu.async_copy(y_sh.at[src_v.at[j]], dst_buf, sem_g)

        def g_wait(dst_buf):
            pltpu.make_async_copy(y_sh.at[src_v.at[0]], dst_buf, sem_g).wait()

        def s_start(j, src_buf):
            pltpu.async_copy(src_buf, acc_sh.at[dst_v.at[j]], sem_s, add=True)

        def s_wait(src_buf):
            pltpu.make_async_copy(
                src_buf, acc_sh.at[dst_v.at[0]], sem_s).wait()

        def hop_group(g, carry):
            pltpu.sync_copy(src_h.at[s, pl.ds(g * GE, GE)], src_v)
            pltpu.sync_copy(dst_h.at[s, pl.ds(g * GE, GE)], dst_v)
            g_start(0, rows_a)
            g_start(1, rows_b)

            def triple(m, carry2):
                j0 = 3 * m
                for kk in range(3):
                    jj = j0 + kk
                    bcur = rings[kk]
                    bprev = rings[(kk + 2) % 3]
                    g_wait(bcur)
                    s_start(jj, bcur)
                    if kk == 0:
                        @pl.when(m > 0)
                        def _():
                            s_wait(bprev)

                        g_start(jj + 2, bprev)
                    else:
                        s_wait(bprev)

                        @pl.when(m < NT - 1)
                        def _():
                            g_start(jj + 2, bprev)
                return carry2

            lax.fori_loop(0, NT, triple, 0)
            s_wait(rows_c)
            return carry

        lax.fori_loop(0, NG, hop_group, 0)
        plsc.subcore_barrier()

        scale_rows(2, False)
        plsc.subcore_barrier()

        lax.fori_loop(0, NG, hop_group, 0)
        plsc.subcore_barrier()

        scale_rows(1, True)
    return k(x2, srcb, dstb)


def _tc_linear(h2, W, b):
    def body(h2_ref, w_ref, b_ref, o_ref):
        h = jnp.concatenate([h2_ref[0], h2_ref[1]], axis=1)[:N]
        o_ref[...] = lax.dot_general(
            h, w_ref[...], (((1,), (1,)), ((), ())),
            preferred_element_type=jnp.float32) + b_ref[...]

    return pl.pallas_call(
        body,
        out_shape=jax.ShapeDtypeStruct((N, D), jnp.float32),
    )(h2, W, b)


@jax.jit
def kernel(x, edge_index, W, b):
    src = edge_index[0].astype(jnp.int32)
    dst = edge_index[1].astype(jnp.int32)
    pad = jnp.full((EP - E,), N, jnp.int32)
    srcb = jnp.concatenate([src, pad]).reshape(16, NCH, CH)
    dstb = jnp.concatenate([dst, pad]).reshape(16, NCH, CH)
    xp = jnp.concatenate([x, jnp.zeros((NP - N, D), x.dtype)])
    x2 = xp.reshape(NP, 2, DH).transpose(1, 0, 2)
    h2 = _sc_propagate(x2, srcb, dstb)
    return _tc_linear(h2, W, b)


# speedup vs baseline: 25.7997x; 1.0973x over previous
import functools

import jax
import jax.numpy as jnp
from jax import lax
from jax.experimental import pallas as pl
from jax.experimental.pallas import tpu as pltpu
from jax.experimental.pallas import tpu_sc as plsc

N = 10000
NP = 10240
D = 128
DH = 64
E = 320000
CH = 128
GB = 9
NGRP = 18
NCH = GB * NGRP
EP = 16 * NCH * CH
RPT = NP // 16
RB = 64
NB = RPT // RB
NTT = NCH // 3


def _sc_propagate(x2, srcb, dstb):
    mesh = plsc.VectorSubcoreMesh(core_axis_name="c", subcore_axis_name="s")

    @functools.partial(
        pl.kernel,
        out_type=jax.ShapeDtypeStruct((2, NP, DH), jnp.float32),
        mesh=mesh,
        compiler_params=pltpu.CompilerParams(use_tc_tiling_on_sc=False),
        scratch_types=[
            pltpu.VMEM((2, GB, CH), jnp.int32),
            pltpu.VMEM((2, GB, CH), jnp.int32),
            pltpu.VMEM((CH, DH), jnp.float32),
            pltpu.VMEM((CH, DH), jnp.float32),
            pltpu.VMEM((CH, DH), jnp.float32),
            pltpu.VMEM((RB, DH), jnp.float32),
            pltpu.VMEM((RB, 16), jnp.float32),
            pltpu.VMEM((CH, 16), jnp.float32),
            pltpu.VMEM_SHARED((NP, DH), jnp.float32),
            pltpu.VMEM_SHARED((NP, DH), jnp.float32),
            pltpu.VMEM_SHARED((NP, 16), jnp.float32),
            pltpu.SemaphoreType.DMA,
            pltpu.SemaphoreType.DMA,
            pltpu.SemaphoreType.DMA,
        ],
    )
    def k(x2_h, src_h, dst_h, out_h, src_v, dst_v, rows_a, rows_b, rows_c,
          buf, dloc, ones_v, y_sh, acc_sh, deg_sh, sem_g, sem_s, sem_i):
        c = lax.axis_index("c")
        s = lax.axis_index("s")
        row0 = s * RPT
        own = pl.ds(row0, RPT)
        rings = (rows_a, rows_b, rows_c)

        z16 = jnp.zeros((16,), jnp.float32)
        o16 = jnp.ones((16,), jnp.float32)

        def z_body(r, carry):
            dloc[r] = z16
            return carry

        lax.fori_loop(0, RB, z_body, 0)

        def o_body(r, carry):
            ones_v[r] = o16
            return carry

        lax.fori_loop(0, CH, o_body, 0)

        def zc_body(cb, carry):
            pltpu.sync_copy(dloc, deg_sh.at[pl.ds(row0 + cb * RB, RB)])
            return carry

        lax.fori_loop(0, NB, zc_body, 0)
        pltpu.sync_copy(x2_h.at[c, own], y_sh.at[own])
        plsc.subcore_barrier()

        def deg_group(g, carry):
            p = lax.rem(g, 2)
            pltpu.sync_copy(dst_h.at[s, pl.ds(g * GB, GB)], dst_v.at[p])

            def deg_fire(j, carry2):
                pltpu.async_copy(ones_v, deg_sh.at[dst_v.at[p, j]], sem_s,
                                 add=True)
                return carry2

            lax.fori_loop(0, GB, deg_fire, 0)

            def deg_drain(j, carry2):
                pltpu.make_async_copy(
                    ones_v, deg_sh.at[dst_v.at[0, 0]], sem_s).wait()
                return carry2

            lax.fori_loop(0, GB, deg_drain, 0)
            return carry

        lax.fori_loop(0, NGRP, deg_group, 0)
        plsc.subcore_barrier()

        def dinv_chunk(cb, carry):
            sl = pl.ds(row0 + cb * RB, RB)
            pltpu.sync_copy(deg_sh.at[sl], dloc)

            def dinv_body(r, carry2):
                d = dloc[r] + 1.0
                i = lax.bitcast_convert_type(d, jnp.int32)
                i = jnp.int32(0x5F3759DF) - (i >> 1)
                q = lax.bitcast_convert_type(i, jnp.float32)
                q = q * (1.5 - 0.5 * d * q * q)
                q = q * (1.5 - 0.5 * d * q * q)
                q = q * (1.5 - 0.5 * d * q * q)
                dloc[r] = q
                return carry2

            lax.fori_loop(0, RB, dinv_body, 0)
            pltpu.sync_copy(dloc, deg_sh.at[sl])
            return carry

        lax.fori_loop(0, NB, dinv_chunk, 0)

        def scale_rows(power, to_out):
            def chunk_body(cb, carry):
                sl = pl.ds(row0 + cb * RB, RB)
                if to_out:
                    src_ref = acc_sh.at[sl]
                else:
                    src_ref = y_sh.at[sl] if power == 1 else acc_sh.at[sl]
                pltpu.sync_copy(src_ref, buf)
                pltpu.sync_copy(deg_sh.at[sl], dloc)

                def row_body(r, carry2):
                    dv = dloc[r]
                    if power == 2:
                        dv = dv * dv
                    for jj in range(DH // 16):
                        csl = pl.ds(jj * 16, 16)
                        buf[r, csl] = buf[r, csl] * dv
                    return carry2

                lax.fori_loop(0, RB, row_body, 0)
                if to_out:
                    pltpu.sync_copy(buf, y_sh.at[sl])
                    pltpu.sync_copy(y_sh.at[sl], out_h.at[c, sl])
                else:
                    pltpu.sync_copy(buf, y_sh.at[sl])
                    pltpu.sync_copy(buf, acc_sh.at[sl])
                return carry

            lax.fori_loop(0, NB, chunk_body, 0)

        scale_rows(1, False)
        plsc.subcore_barrier()

        def sidx(j):
            return lax.rem(lax.div(j, GB), 2), lax.rem(j, GB)

        def i_start(gidx):
            p = lax.rem(gidx, 2)
            sl = pl.ds(gidx * GB, GB)
            pltpu.async_copy(src_h.at[s, sl], src_v.at[p], sem_i)
            pltpu.async_copy(dst_h.at[s, sl], dst_v.at[p], sem_i)

        def i_wait():
            pltpu.make_async_copy(
                src_h.at[s, pl.ds(0, GB)], src_v.at[0], sem_i).wait()
            pltpu.make_async_copy(
                dst_h.at[s, pl.ds(0, GB)], dst_v.at[0], sem_i).wait()

        def g_start(j, dst_buf):
            p, r = sidx(j)
            pltpu.async_copy(y_sh.at[src_v.at[p, r]], dst_buf, sem_g)

        def g_wait(dst_buf):
            pltpu.make_async_copy(
                y_sh.at[src_v.at[0, 0]], dst_buf, sem_g).wait()

        def s_start(j, src_buf):
            p, r = sidx(j)
            pltpu.async_copy(src_buf, acc_sh.at[dst_v.at[p, r]], sem_s,
                             add=True)

        def s_wait(src_buf):
            pltpu.make_async_copy(
                src_buf, acc_sh.at[dst_v.at[0, 0]], sem_s).wait()

        def hop():
            pltpu.sync_copy(src_h.at[s, pl.ds(0, GB)], src_v.at[0])
            pltpu.sync_copy(dst_h.at[s, pl.ds(0, GB)], dst_v.at[0])
            i_start(1)
            g_start(0, rows_a)
            g_start(1, rows_b)

            def triple(m, carry2):
                j0 = 3 * m
                for kk in range(3):
                    jj = j0 + kk
                    bcur = rings[kk]
                    bprev = rings[(kk + 2) % 3]
                    g_wait(bcur)
                    s_start(jj, bcur)
                    if kk == 0:
                        @pl.when(m > 0)
                        def _():
                            s_wait(bprev)

                        gnext = lax.div(jj, GB) + 1

                        @pl.when((lax.rem(jj, GB) == 0) & (gnext < NGRP))
                        def _():
                            i_start(gnext)

                        g_start(jj + 2, bprev)
                    else:
                        s_wait(bprev)
                        jf = jj + 2

                        @pl.when((lax.rem(jf, GB) == 0) & (jf < NCH))
                        def _():
                            i_wait()

                        @pl.when(jf < NCH)
                        def _():
                            g_start(jf, bprev)
                return carry2

            lax.fori_loop(0, NTT, triple, 0)
            s_wait(rows_c)

        hop()
        plsc.subcore_barrier()

        scale_rows(2, False)
        plsc.subcore_barrier()

        hop()
        plsc.subcore_barrier()

        scale_rows(1, True)
    return k(x2, srcb, dstb)


def _tc_linear(h2, W, b):
    def body(h2_ref, w_ref, b_ref, o_ref):
        h = jnp.concatenate([h2_ref[0], h2_ref[1]], axis=1)[:N]
        o_ref[...] = lax.dot_general(
            h, w_ref[...], (((1,), (1,)), ((), ())),
            preferred_element_type=jnp.float32) + b_ref[...]

    return pl.pallas_call(
        body,
        out_shape=jax.ShapeDtypeStruct((N, D), jnp.float32),
    )(h2, W, b)


@jax.jit
def kernel(x, edge_index, W, b):
    src = edge_index[0].astype(jnp.int32)
    dst = edge_index[1].astype(jnp.int32)
    pad = jnp.full((EP - E,), N, jnp.int32)
    srcb = jnp.concatenate([src, pad]).reshape(16, NCH, CH)
    dstb = jnp.concatenate([dst, pad]).reshape(16, NCH, CH)
    xp = jnp.concatenate([x, jnp.zeros((NP - N, D), x.dtype)])
    x2 = xp.reshape(NP, 2, DH).transpose(1, 0, 2)
    h2 = _sc_propagate(x2, srcb, dstb)
    return _tc_linear(h2, W, b)


# speedup vs baseline: 27.0908x; 1.0500x over previous
import functools

import jax
import jax.numpy as jnp
from jax import lax
from jax.experimental import pallas as pl
from jax.experimental.pallas import tpu as pltpu
from jax.experimental.pallas import tpu_sc as plsc

N = 10000
NP = 10240
D = 128
DH = 64
E = 320000
CH = 128
GB = 9
NGRP = 18
NCH = GB * NGRP
EP = 16 * NCH * CH
RPT = NP // 16
RB = 64
NB = RPT // RB
NTT = NCH // 3


def _sc_propagate(x2, srcb, dstb):
    mesh = plsc.VectorSubcoreMesh(core_axis_name="c", subcore_axis_name="s")

    @functools.partial(
        pl.kernel,
        out_type=jax.ShapeDtypeStruct((2, NP, DH), jnp.float32),
        mesh=mesh,
        compiler_params=pltpu.CompilerParams(use_tc_tiling_on_sc=False),
        scratch_types=[
            pltpu.VMEM((2, GB, CH), jnp.int32),
            pltpu.VMEM((2, GB, CH), jnp.int32),
            pltpu.VMEM((CH, DH), jnp.float32),
            pltpu.VMEM((CH, DH), jnp.float32),
            pltpu.VMEM((CH, DH), jnp.float32),
            pltpu.VMEM((RB, DH), jnp.float32),
            pltpu.VMEM((RB, 16), jnp.float32),
            pltpu.VMEM((CH, 16), jnp.float32),
            pltpu.VMEM_SHARED((NP, DH), jnp.float32),
            pltpu.VMEM_SHARED((NP, DH), jnp.float32),
            pltpu.VMEM_SHARED((NP, 16), jnp.float32),
            pltpu.SemaphoreType.DMA,
            pltpu.SemaphoreType.DMA,
            pltpu.SemaphoreType.DMA,
        ],
    )
    def k(x2_h, src_h, dst_h, out_h, src_v, dst_v, rows_a, rows_b, rows_c,
          buf, dloc, ones_v, y_sh, acc_sh, deg_sh, sem_g, sem_s, sem_i):
        c = lax.axis_index("c")
        s = lax.axis_index("s")
        row0 = s * RPT
        own = pl.ds(row0, RPT)
        rings = (rows_a, rows_b, rows_c)

        z16 = jnp.zeros((16,), jnp.float32)
        o16 = jnp.ones((16,), jnp.float32)

        def z_body(r, carry):
            dloc[r] = z16
            return carry

        lax.fori_loop(0, RB, z_body, 0)

        def o_body(r, carry):
            ones_v[r] = o16
            return carry

        lax.fori_loop(0, CH, o_body, 0)

        def zc_body(cb, carry):
            pltpu.sync_copy(dloc, deg_sh.at[pl.ds(row0 + cb * RB, RB)])
            return carry

        lax.fori_loop(0, NB, zc_body, 0)
        pltpu.sync_copy(x2_h.at[own, pl.ds(c * DH, DH)], y_sh.at[own])
        plsc.subcore_barrier()

        def deg_group(g, carry):
            p = lax.rem(g, 2)
            pltpu.sync_copy(dst_h.at[s, pl.ds(g * GB, GB)], dst_v.at[p])

            def deg_fire(j, carry2):
                pltpu.async_copy(ones_v, deg_sh.at[dst_v.at[p, j]], sem_s,
                                 add=True)
                return carry2

            lax.fori_loop(0, GB, deg_fire, 0)

            def deg_drain(j, carry2):
                pltpu.make_async_copy(
                    ones_v, deg_sh.at[dst_v.at[0, 0]], sem_s).wait()
                return carry2

            lax.fori_loop(0, GB, deg_drain, 0)
            return carry

        lax.fori_loop(0, NGRP, deg_group, 0)
        plsc.subcore_barrier()

        def dinv_chunk(cb, carry):
            sl = pl.ds(row0 + cb * RB, RB)
            pltpu.sync_copy(deg_sh.at[sl], dloc)

            def dinv_body(r, carry2):
                d = dloc[r] + 1.0
                i = lax.bitcast_convert_type(d, jnp.int32)
                i = jnp.int32(0x5F3759DF) - (i >> 1)
                q = lax.bitcast_convert_type(i, jnp.float32)
                q = q * (1.5 - 0.5 * d * q * q)
                q = q * (1.5 - 0.5 * d * q * q)
                q = q * (1.5 - 0.5 * d * q * q)
                dloc[r] = q
                return carry2

            lax.fori_loop(0, RB, dinv_body, 0)
            pltpu.sync_copy(dloc, deg_sh.at[sl])
            return carry

        lax.fori_loop(0, NB, dinv_chunk, 0)

        def scale_rows(power, to_out):
            def chunk_body(cb, carry):
                sl = pl.ds(row0 + cb * RB, RB)
                if to_out:
                    src_ref = acc_sh.at[sl]
                else:
                    src_ref = y_sh.at[sl] if power == 1 else acc_sh.at[sl]
                pltpu.sync_copy(src_ref, buf)
                pltpu.sync_copy(deg_sh.at[sl], dloc)

                def row_body(r, carry2):
                    dv = dloc[r]
                    if power == 2:
                        dv = dv * dv
                    for jj in range(DH // 16):
                        csl = pl.ds(jj * 16, 16)
                        buf[r, csl] = buf[r, csl] * dv
                    return carry2

                lax.fori_loop(0, RB, row_body, 0)
                if to_out:
                    pltpu.sync_copy(buf, y_sh.at[sl])
                    pltpu.sync_copy(y_sh.at[sl], out_h.at[c, sl])
                else:
                    pltpu.sync_copy(buf, y_sh.at[sl])
                    pltpu.sync_copy(buf, acc_sh.at[sl])
                return carry

            lax.fori_loop(0, NB, chunk_body, 0)

        scale_rows(1, False)
        plsc.subcore_barrier()

        def sidx(j):
            return lax.rem(lax.div(j, GB), 2), lax.rem(j, GB)

        def i_start(gidx):
            p = lax.rem(gidx, 2)
            sl = pl.ds(gidx * GB, GB)
            pltpu.async_copy(src_h.at[s, sl], src_v.at[p], sem_i)
            pltpu.async_copy(dst_h.at[s, sl], dst_v.at[p], sem_i)

        def i_wait():
            pltpu.make_async_copy(
                src_h.at[s, pl.ds(0, GB)], src_v.at[0], sem_i).wait()
            pltpu.make_async_copy(
                dst_h.at[s, pl.ds(0, GB)], dst_v.at[0], sem_i).wait()

        def g_start(j, dst_buf):
            p, r = sidx(j)
            pltpu.async_copy(y_sh.at[src_v.at[p, r]], dst_buf, sem_g)

        def g_wait(dst_buf):
            pltpu.make_async_copy(
                y_sh.at[src_v.at[0, 0]], dst_buf, sem_g).wait()

        def s_start(j, src_buf):
            p, r = sidx(j)
            pltpu.async_copy(src_buf, acc_sh.at[dst_v.at[p, r]], sem_s,
                             add=True)

        def s_wait(src_buf):
            pltpu.make_async_copy(
                src_buf, acc_sh.at[dst_v.at[0, 0]], sem_s).wait()

        def hop():
            pltpu.sync_copy(src_h.at[s, pl.ds(0, GB)], src_v.at[0])
            pltpu.sync_copy(dst_h.at[s, pl.ds(0, GB)], dst_v.at[0])
            i_start(1)
            g_start(0, rows_a)
            g_start(1, rows_b)

            def triple(m, carry2):
                j0 = 3 * m
                for kk in range(3):
                    jj = j0 + kk
                    bcur = rings[kk]
                    bprev = rings[(kk + 2) % 3]
                    g_wait(bcur)
                    s_start(jj, bcur)
                    if kk == 0:
                        @pl.when(m > 0)
                        def _():
                            s_wait(bprev)

                        gnext = lax.div(jj, GB) + 1

                        @pl.when((lax.rem(jj, GB) == 0) & (gnext < NGRP))
                        def _():
                            i_start(gnext)

                        g_start(jj + 2, bprev)
                    else:
                        s_wait(bprev)
                        jf = jj + 2

                        @pl.when((lax.rem(jf, GB) == 0) & (jf < NCH))
                        def _():
                            i_wait()

                        @pl.when(jf < NCH)
                        def _():
                            g_start(jf, bprev)
                return carry2

            lax.fori_loop(0, NTT, triple, 0)
            s_wait(rows_c)

        hop()
        plsc.subcore_barrier()

        scale_rows(2, False)
        plsc.subcore_barrier()

        hop()
        plsc.subcore_barrier()

        scale_rows(1, True)
    return k(x2, srcb, dstb)


def _tc_linear(h2, W, b):
    def body(h2_ref, w_ref, b_ref, o_ref):
        h = jnp.concatenate([h2_ref[0], h2_ref[1]], axis=1)[:N]
        o_ref[...] = lax.dot_general(
            h, w_ref[...], (((1,), (1,)), ((), ())),
            preferred_element_type=jnp.float32) + b_ref[...]

    return pl.pallas_call(
        body,
        out_shape=jax.ShapeDtypeStruct((N, D), jnp.float32),
    )(h2, W, b)


@jax.jit
def kernel(x, edge_index, W, b):
    src = edge_index[0].astype(jnp.int32)
    dst = edge_index[1].astype(jnp.int32)
    pad = jnp.full((EP - E,), N, jnp.int32)
    srcb = jnp.concatenate([src, pad]).reshape(16, NCH, CH)
    dstb = jnp.concatenate([dst, pad]).reshape(16, NCH, CH)
    xp = jnp.concatenate([x, jnp.zeros((NP - N, D), x.dtype)])
    h2 = _sc_propagate(xp, srcb, dstb)
    return _tc_linear(h2, W, b)


# speedup vs baseline: 27.9562x; 1.0319x over previous
import functools

import jax
import jax.numpy as jnp
from jax import lax
from jax.experimental import pallas as pl
from jax.experimental.pallas import tpu as pltpu
from jax.experimental.pallas import tpu_sc as plsc

N = 10000
NP = 10240
D = 128
DH = 64
E = 320000
CH = 128
GB = 9
NGRP = 18
NCH = GB * NGRP
EP = 16 * NCH * CH
RPT = NP // 16
RB = 64
NB = RPT // RB
NTT = NCH // 3


def _sc_propagate(x2, srcb, dstb):
    mesh = plsc.VectorSubcoreMesh(core_axis_name="c", subcore_axis_name="s")

    @functools.partial(
        pl.kernel,
        out_type=[
            jax.ShapeDtypeStruct((2, NP, DH), jnp.float32),
            jax.ShapeDtypeStruct((NP, 16), jnp.float32),
        ],
        mesh=mesh,
        compiler_params=pltpu.CompilerParams(use_tc_tiling_on_sc=False),
        scratch_types=[
            pltpu.VMEM((2, GB, CH), jnp.int32),
            pltpu.VMEM((2, GB, CH), jnp.int32),
            pltpu.VMEM((CH, DH), jnp.float32),
            pltpu.VMEM((CH, DH), jnp.float32),
            pltpu.VMEM((CH, DH), jnp.float32),
            pltpu.VMEM((RB, DH), jnp.float32),
            pltpu.VMEM((RB, 16), jnp.float32),
            pltpu.VMEM((CH, 16), jnp.float32),
            pltpu.VMEM_SHARED((NP, DH), jnp.float32),
            pltpu.VMEM_SHARED((NP, DH), jnp.float32),
            pltpu.VMEM_SHARED((NP, 16), jnp.float32),
            pltpu.SemaphoreType.DMA,
            pltpu.SemaphoreType.DMA,
            pltpu.SemaphoreType.DMA,
        ],
    )
    def k(x2_h, src_h, dst_h, out_h, dinv_h, src_v, dst_v, rows_a, rows_b,
          rows_c, buf, dloc, ones_v, y_sh, acc_sh, deg_sh, sem_g, sem_s,
          sem_i):
        c = lax.axis_index("c")
        s = lax.axis_index("s")
        row0 = s * RPT
        own = pl.ds(row0, RPT)
        rings = (rows_a, rows_b, rows_c)

        z16 = jnp.zeros((16,), jnp.float32)
        o16 = jnp.ones((16,), jnp.float32)

        def z_body(r, carry):
            dloc[r] = z16
            return carry

        lax.fori_loop(0, RB, z_body, 0)

        def o_body(r, carry):
            ones_v[r] = o16
            return carry

        lax.fori_loop(0, CH, o_body, 0)

        def zc_body(cb, carry):
            pltpu.sync_copy(dloc, deg_sh.at[pl.ds(row0 + cb * RB, RB)])
            return carry

        lax.fori_loop(0, NB, zc_body, 0)
        pltpu.sync_copy(x2_h.at[own, pl.ds(c * DH, DH)], y_sh.at[own])
        plsc.subcore_barrier()

        pltpu.sync_copy(dst_h.at[s, pl.ds(0, GB)], dst_v.at[0])

        def deg_group(g, carry):
            p = lax.rem(g, 2)

            @pl.when(g > 0)
            def _():
                pltpu.make_async_copy(
                    dst_h.at[s, pl.ds(0, GB)], dst_v.at[0], sem_i).wait()

            @pl.when(g + 1 < NGRP)
            def _():
                pltpu.async_copy(dst_h.at[s, pl.ds((g + 1) * GB, GB)],
                                 dst_v.at[lax.rem(g + 1, 2)], sem_i)

            def deg_fire(j, carry2):
                pltpu.async_copy(ones_v, deg_sh.at[dst_v.at[p, j]], sem_s,
                                 add=True)
                return carry2

            lax.fori_loop(0, GB, deg_fire, 0)

            def deg_drain(j, carry2):
                pltpu.make_async_copy(
                    ones_v, deg_sh.at[dst_v.at[0, 0]], sem_s).wait()
                return carry2

            lax.fori_loop(0, GB, deg_drain, 0)
            return carry

        lax.fori_loop(0, NGRP, deg_group, 0)
        plsc.subcore_barrier()

        def dinv_chunk(cb, carry):
            sl = pl.ds(row0 + cb * RB, RB)
            pltpu.sync_copy(deg_sh.at[sl], dloc)

            def dinv_body(r, carry2):
                d = dloc[r] + 1.0
                i = lax.bitcast_convert_type(d, jnp.int32)
                i = jnp.int32(0x5F3759DF) - (i >> 1)
                q = lax.bitcast_convert_type(i, jnp.float32)
                q = q * (1.5 - 0.5 * d * q * q)
                q = q * (1.5 - 0.5 * d * q * q)
                q = q * (1.5 - 0.5 * d * q * q)
                dloc[r] = q
                return carry2

            lax.fori_loop(0, RB, dinv_body, 0)
            pltpu.sync_copy(dloc, deg_sh.at[sl])
            return carry

        lax.fori_loop(0, NB, dinv_chunk, 0)

        def scale_rows(power):
            def chunk_body(cb, carry):
                sl = pl.ds(row0 + cb * RB, RB)
                src_ref = y_sh.at[sl] if power == 1 else acc_sh.at[sl]
                pltpu.sync_copy(src_ref, buf)
                pltpu.sync_copy(deg_sh.at[sl], dloc)

                def row_body(r, carry2):
                    dv = dloc[r]
                    if power == 2:
                        dv = dv * dv
                    for jj in range(DH // 16):
                        csl = pl.ds(jj * 16, 16)
                        buf[r, csl] = buf[r, csl] * dv
                    return carry2

                lax.fori_loop(0, RB, row_body, 0)
                pltpu.sync_copy(buf, y_sh.at[sl])
                pltpu.sync_copy(buf, acc_sh.at[sl])
                return carry

            lax.fori_loop(0, NB, chunk_body, 0)

        scale_rows(1)
        plsc.subcore_barrier()

        def sidx(j):
            return lax.rem(lax.div(j, GB), 2), lax.rem(j, GB)

        def i_start(gidx):
            p = lax.rem(gidx, 2)
            sl = pl.ds(gidx * GB, GB)
            pltpu.async_copy(src_h.at[s, sl], src_v.at[p], sem_i)
            pltpu.async_copy(dst_h.at[s, sl], dst_v.at[p], sem_i)

        def i_wait():
            pltpu.make_async_copy(
                src_h.at[s, pl.ds(0, GB)], src_v.at[0], sem_i).wait()
            pltpu.make_async_copy(
                dst_h.at[s, pl.ds(0, GB)], dst_v.at[0], sem_i).wait()

        def g_start(j, dst_buf):
            p, r = sidx(j)
            pltpu.async_copy(y_sh.at[src_v.at[p, r]], dst_buf, sem_g)

        def g_wait(dst_buf):
            pltpu.make_async_copy(
                y_sh.at[src_v.at[0, 0]], dst_buf, sem_g).wait()

        def s_start(j, src_buf):
            p, r = sidx(j)
            pltpu.async_copy(src_buf, acc_sh.at[dst_v.at[p, r]], sem_s,
                             add=True)

        def s_wait(src_buf):
            pltpu.make_async_copy(
                src_buf, acc_sh.at[dst_v.at[0, 0]], sem_s).wait()

        def hop():
            pltpu.sync_copy(src_h.at[s, pl.ds(0, GB)], src_v.at[0])
            pltpu.sync_copy(dst_h.at[s, pl.ds(0, GB)], dst_v.at[0])
            i_start(1)
            g_start(0, rows_a)
            g_start(1, rows_b)

            def triple(m, carry2):
                j0 = 3 * m
                for kk in range(3):
                    jj = j0 + kk
                    bcur = rings[kk]
                    bprev = rings[(kk + 2) % 3]
                    g_wait(bcur)
                    s_start(jj, bcur)
                    if kk == 0:
                        @pl.when(m > 0)
                        def _():
                            s_wait(bprev)

                        gnext = lax.div(jj, GB) + 1

                        @pl.when((lax.rem(jj, GB) == 0) & (gnext < NGRP))
                        def _():
                            i_start(gnext)

                        g_start(jj + 2, bprev)
                    else:
                        s_wait(bprev)
                        jf = jj + 2

                        @pl.when((lax.rem(jf, GB) == 0) & (jf < NCH))
                        def _():
                            i_wait()

                        @pl.when(jf < NCH)
                        def _():
                            g_start(jf, bprev)
                return carry2

            lax.fori_loop(0, NTT, triple, 0)
            s_wait(rows_c)

        hop()
        plsc.subcore_barrier()

        scale_rows(2)
        plsc.subcore_barrier()

        hop()
        plsc.subcore_barrier()

        pltpu.sync_copy(acc_sh.at[own], out_h.at[c, own])

        @pl.when(c == 0)
        def _():
            pltpu.sync_copy(deg_sh.at[own], dinv_h.at[own])

    return k(x2, srcb, dstb)


def _tc_linear(h2, dinvs, W, b):
    def body(h2_ref, dinv_ref, w_ref, b_ref, o_ref):
        h = jnp.concatenate([h2_ref[0], h2_ref[1]], axis=1)[:N]
        h = h * dinv_ref[:N, 0:1]
        o_ref[...] = lax.dot_general(
            h, w_ref[...], (((1,), (1,)), ((), ())),
            preferred_element_type=jnp.float32) + b_ref[...]

    return pl.pallas_call(
        body,
        out_shape=jax.ShapeDtypeStruct((N, D), jnp.float32),
    )(h2, dinvs, W, b)


@jax.jit
def kernel(x, edge_index, W, b):
    src = edge_index[0].astype(jnp.int32)
    dst = edge_index[1].astype(jnp.int32)
    pad = jnp.full((EP - E,), N, jnp.int32)
    srcb = jnp.concatenate([src, pad]).reshape(16, NCH, CH)
    dstb = jnp.concatenate([dst, pad]).reshape(16, NCH, CH)
    xp = jnp.concatenate([x, jnp.zeros((NP - N, D), x.dtype)])
    h2, dinvs = _sc_propagate(xp, srcb, dstb)
    return _tc_linear(h2, dinvs, W, b)


# speedup vs baseline: 28.3944x; 1.0157x over previous
import functools

import jax
import jax.numpy as jnp
from jax import lax
from jax.experimental import pallas as pl
from jax.experimental.pallas import tpu as pltpu
from jax.experimental.pallas import tpu_sc as plsc

N = 10000
NP = 10240
D = 128
DH = 64
E = 320000
CH = 128
GB = 9
NGRP = 18
NCH = GB * NGRP
EP = 16 * NCH * CH
RPT = NP // 16
RB = 32
NB = RPT // RB
NTT = NCH // 3


def _sc_propagate(x2, srcb, dstb):
    mesh = plsc.VectorSubcoreMesh(core_axis_name="c", subcore_axis_name="s")

    @functools.partial(
        pl.kernel,
        out_type=[
            jax.ShapeDtypeStruct((2, NP, DH), jnp.float32),
            jax.ShapeDtypeStruct((NP, 16), jnp.float32),
        ],
        mesh=mesh,
        compiler_params=pltpu.CompilerParams(use_tc_tiling_on_sc=False),
        scratch_types=[
            pltpu.VMEM((2, GB, CH), jnp.int32),
            pltpu.VMEM((2, GB, CH), jnp.int32),
            pltpu.VMEM((CH, DH), jnp.float32),
            pltpu.VMEM((CH, DH), jnp.float32),
            pltpu.VMEM((CH, DH), jnp.float32),
            pltpu.VMEM((2, RB, DH), jnp.float32),
            pltpu.VMEM((2, RB, 16), jnp.float32),
            pltpu.VMEM((CH, 16), jnp.float32),
            pltpu.VMEM_SHARED((NP, DH), jnp.float32),
            pltpu.VMEM_SHARED((NP, DH), jnp.float32),
            pltpu.VMEM_SHARED((NP, 16), jnp.float32),
            pltpu.SemaphoreType.DMA,
            pltpu.SemaphoreType.DMA,
            pltpu.SemaphoreType.DMA,
        ],
    )
    def k(x2_h, src_h, dst_h, out_h, dinv_h, src_v, dst_v, rows_a, rows_b,
          rows_c, buf, dloc, ones_v, y_sh, acc_sh, deg_sh, sem_g, sem_s,
          sem_i):
        c = lax.axis_index("c")
        s = lax.axis_index("s")
        row0 = s * RPT
        own = pl.ds(row0, RPT)
        rings = (rows_a, rows_b, rows_c)

        z16 = jnp.zeros((16,), jnp.float32)
        o16 = jnp.ones((16,), jnp.float32)

        def z_body(r, carry):
            dloc[0, r] = z16
            return carry

        lax.fori_loop(0, RB, z_body, 0)

        def o_body(r, carry):
            ones_v[r] = o16
            return carry

        lax.fori_loop(0, CH, o_body, 0)

        def zc_body(cb, carry):
            pltpu.sync_copy(dloc.at[0],
                            deg_sh.at[pl.ds(row0 + cb * RB, RB)])
            return carry

        lax.fori_loop(0, NB, zc_body, 0)
        pltpu.sync_copy(x2_h.at[own, pl.ds(c * DH, DH)], y_sh.at[own])
        plsc.subcore_barrier()

        pltpu.sync_copy(dst_h.at[s, pl.ds(0, GB)], dst_v.at[0])

        def deg_group(g, carry):
            p = lax.rem(g, 2)

            @pl.when(g > 0)
            def _():
                pltpu.make_async_copy(
                    dst_h.at[s, pl.ds(0, GB)], dst_v.at[0], sem_i).wait()

            @pl.when(g + 1 < NGRP)
            def _():
                pltpu.async_copy(dst_h.at[s, pl.ds((g + 1) * GB, GB)],
                                 dst_v.at[lax.rem(g + 1, 2)], sem_i)

            def deg_fire(j, carry2):
                pltpu.async_copy(ones_v, deg_sh.at[dst_v.at[p, j]], sem_s,
                                 add=True)
                return carry2

            lax.fori_loop(0, GB, deg_fire, 0)

            def deg_drain(j, carry2):
                pltpu.make_async_copy(
                    ones_v, deg_sh.at[dst_v.at[0, 0]], sem_s).wait()
                return carry2

            lax.fori_loop(0, GB, deg_drain, 0)
            return carry

        lax.fori_loop(0, NGRP, deg_group, 0)
        plsc.subcore_barrier()

        def dinv_chunk(cb, carry):
            sl = pl.ds(row0 + cb * RB, RB)
            pltpu.sync_copy(deg_sh.at[sl], dloc.at[0])

            def dinv_body(r, carry2):
                d = dloc[0, r] + 1.0
                i = lax.bitcast_convert_type(d, jnp.int32)
                i = jnp.int32(0x5F3759DF) - (i >> 1)
                q = lax.bitcast_convert_type(i, jnp.float32)
                q = q * (1.5 - 0.5 * d * q * q)
                q = q * (1.5 - 0.5 * d * q * q)
                q = q * (1.5 - 0.5 * d * q * q)
                dloc[0, r] = q
                return carry2

            lax.fori_loop(0, RB, dinv_body, 0)
            pltpu.sync_copy(dloc.at[0], deg_sh.at[sl])
            return carry

        lax.fori_loop(0, NB, dinv_chunk, 0)

        def scale_rows(power):
            def src_ref(sl):
                return y_sh.at[sl] if power == 1 else acc_sh.at[sl]

            def ld_start(cb, q):
                sl = pl.ds(row0 + cb * RB, RB)
                pltpu.async_copy(src_ref(sl), buf.at[q], sem_i)
                pltpu.async_copy(deg_sh.at[sl], dloc.at[q], sem_i)

            def ld_wait():
                sl0 = pl.ds(row0, RB)
                pltpu.make_async_copy(src_ref(sl0), buf.at[0], sem_i).wait()
                pltpu.make_async_copy(deg_sh.at[sl0], dloc.at[0],
                                      sem_i).wait()

            def st_wait():
                sl0 = pl.ds(row0, RB)
                pltpu.make_async_copy(buf.at[0], y_sh.at[sl0], sem_s).wait()
                pltpu.make_async_copy(buf.at[0], acc_sh.at[sl0],
                                      sem_s).wait()

            ld_start(0, 0)

            def chunk_body(cb, carry):
                q = lax.rem(cb, 2)
                sl = pl.ds(row0 + cb * RB, RB)
                ld_wait()

                @pl.when(cb > 0)
                def _():
                    st_wait()

                @pl.when(cb + 1 < NB)
                def _():
                    ld_start(cb + 1, 1 - q)

                def row_body(r, carry2):
                    dv = dloc[q, r]
                    if power == 2:
                        dv = dv * dv
                    for jj in range(DH // 16):
                        csl = pl.ds(jj * 16, 16)
                        buf[q, r, csl] = buf[q, r, csl] * dv
                    return carry2

                lax.fori_loop(0, RB, row_body, 0)
                pltpu.async_copy(buf.at[q], y_sh.at[sl], sem_s)
                pltpu.async_copy(buf.at[q], acc_sh.at[sl], sem_s)
                return carry

            lax.fori_loop(0, NB, chunk_body, 0)
            st_wait()

        scale_rows(1)
        plsc.subcore_barrier()

        def sidx(j):
            return lax.rem(lax.div(j, GB), 2), lax.rem(j, GB)

        def i_start(gidx):
            p = lax.rem(gidx, 2)
            sl = pl.ds(gidx * GB, GB)
            pltpu.async_copy(src_h.at[s, sl], src_v.at[p], sem_i)
            pltpu.async_copy(dst_h.at[s, sl], dst_v.at[p], sem_i)

        def i_wait():
            pltpu.make_async_copy(
                src_h.at[s, pl.ds(0, GB)], src_v.at[0], sem_i).wait()
            pltpu.make_async_copy(
                dst_h.at[s, pl.ds(0, GB)], dst_v.at[0], sem_i).wait()

        def g_start(j, dst_buf):
            p, r = sidx(j)
            pltpu.async_copy(y_sh.at[src_v.at[p, r]], dst_buf, sem_g)

        def g_wait(dst_buf):
            pltpu.make_async_copy(
                y_sh.at[src_v.at[0, 0]], dst_buf, sem_g).wait()

        def s_start(j, src_buf):
            p, r = sidx(j)
            pltpu.async_copy(src_buf, acc_sh.at[dst_v.at[p, r]], sem_s,
                             add=True)

        def s_wait(src_buf):
            pltpu.make_async_copy(
                src_buf, acc_sh.at[dst_v.at[0, 0]], sem_s).wait()

        def hop():
            pltpu.sync_copy(src_h.at[s, pl.ds(0, GB)], src_v.at[0])
            pltpu.sync_copy(dst_h.at[s, pl.ds(0, GB)], dst_v.at[0])
            i_start(1)
            g_start(0, rows_a)
            g_start(1, rows_b)

            def triple(m, carry2):
                j0 = 3 * m
                for kk in range(3):
                    jj = j0 + kk
                    bcur = rings[kk]
                    bprev = rings[(kk + 2) % 3]
                    g_wait(bcur)
                    s_start(jj, bcur)
                    if kk == 0:
                        @pl.when(m > 0)
                        def _():
                            s_wait(bprev)

                        gnext = lax.div(jj, GB) + 1

                        @pl.when((lax.rem(jj, GB) == 0) & (gnext < NGRP))
                        def _():
                            i_start(gnext)

                        g_start(jj + 2, bprev)
                    else:
                        s_wait(bprev)
                        jf = jj + 2

                        @pl.when((lax.rem(jf, GB) == 0) & (jf < NCH))
                        def _():
                            i_wait()

                        @pl.when(jf < NCH)
                        def _():
                            g_start(jf, bprev)
                return carry2

            lax.fori_loop(0, NTT, triple, 0)
            s_wait(rows_c)

        hop()
        plsc.subcore_barrier()

        scale_rows(2)
        plsc.subcore_barrier()

        hop()
        plsc.subcore_barrier()

        pltpu.sync_copy(acc_sh.at[own], out_h.at[c, own])

        @pl.when(c == 0)
        def _():
            pltpu.sync_copy(deg_sh.at[own], dinv_h.at[own])

    return k(x2, srcb, dstb)


def _tc_linear(h2, dinvs, W, b):
    def body(h2_ref, dinv_ref, w_ref, b_ref, o_ref):
        h = jnp.concatenate([h2_ref[0], h2_ref[1]], axis=1)[:N]
        h = h * dinv_ref[:N, 0:1]
        o_ref[...] = lax.dot_general(
            h, w_ref[...], (((1,), (1,)), ((), ())),
            preferred_element_type=jnp.float32) + b_ref[...]

    return pl.pallas_call(
        body,
        out_shape=jax.ShapeDtypeStruct((N, D), jnp.float32),
    )(h2, dinvs, W, b)


@jax.jit
def kernel(x, edge_index, W, b):
    src = edge_index[0].astype(jnp.int32)
    dst = edge_index[1].astype(jnp.int32)
    pad = jnp.full((EP - E,), N, jnp.int32)
    srcb = jnp.concatenate([src, pad]).reshape(16, NCH, CH)
    dstb = jnp.concatenate([dst, pad]).reshape(16, NCH, CH)
    xp = jnp.concatenate([x, jnp.zeros((NP - N, D), x.dtype)])
    h2, dinvs = _sc_propagate(xp, srcb, dstb)
    return _tc_linear(h2, dinvs, W, b)
